# Initial kernel scaffold; baseline (speedup 1.0000x reference)
import jax
import jax.numpy as jnp
from jax.experimental import pallas as pl


def kernel(x, emb_table, W, b):
    raise NotImplementedError("write your pallas kernel here")


# speedup vs baseline: 7.3440x; 7.3440x over previous
import functools

import jax
import jax.numpy as jnp
from jax import lax
from jax.experimental import pallas as pl
from jax.experimental.pallas import tpu as pltpu
from jax.experimental.pallas import tpu_sc as plsc

_LANES = 16
_VOCAB_BLK = 5000


def _scores_body(w_ref, b_ref, t_ref, o_ref, *, inv_hist):
    t = t_ref[...]
    w = w_ref[...]
    s = jnp.sum(t * w, axis=1, keepdims=True)
    o_ref[...] = (s + b_ref[0]) * inv_hist


def _scores(emb_table, W, b, hist):
    V, D = emb_table.shape
    grid = V // _VOCAB_BLK
    return pl.pallas_call(
        functools.partial(_scores_body, inv_hist=1.0 / float(hist)),
        grid=(grid,),
        in_specs=[
            pl.BlockSpec((1, D), lambda i: (0, 0)),
            pl.BlockSpec(memory_space=pltpu.SMEM),
            pl.BlockSpec((_VOCAB_BLK, D), lambda i: (i, 0)),
        ],
        out_specs=pl.BlockSpec((_VOCAB_BLK, 1), lambda i: (i, 0)),
        out_shape=jax.ShapeDtypeStruct((V, 1), jnp.float32),
    )(W, b, emb_table)


def _round4(y):
    t = y * 10000.0
    t = (t + 8388608.0) - 8388608.0
    return t / 10000.0


def _pool(x_T, s_flat, batch, hist):
    info = plsc.get_sparse_core_info()
    nc, ns = info.num_cores, info.num_subcores
    nw = nc * ns
    b_per = batch // nw
    groups = b_per // _LANES

    mesh = plsc.VectorSubcoreMesh(core_axis_name="c", subcore_axis_name="s")

    @functools.partial(
        pl.kernel,
        out_type=jax.ShapeDtypeStruct((batch,), jnp.float32),
        mesh=mesh,
        scratch_types=[
            pltpu.VMEM((hist, b_per), jnp.int32),
            pltpu.VMEM((hist, b_per), jnp.float32),
            pltpu.VMEM((b_per,), jnp.float32),
            pltpu.SemaphoreType.DMA,
        ],
    )
    def run(xT_hbm, s_hbm, out_hbm, idx_t, vals_t, y_v, sem):
        wid = lax.axis_index("s") * nc + lax.axis_index("c")
        base = wid * b_per
        pltpu.sync_copy(xT_hbm.at[:, pl.ds(base, b_per)], idx_t)

        n_chunk = 10

        def gather_chunk(k, carry):
            descs = [
                pltpu.async_copy(
                    s_hbm.at[idx_t.at[k * n_chunk + c]],
                    vals_t.at[k * n_chunk + c],
                    sem,
                )
                for c in range(n_chunk)
            ]
            for d in descs:
                d.wait()
            return carry

        lax.fori_loop(0, hist // n_chunk, gather_chunk, 0)

        zero = jnp.zeros((_LANES,), jnp.float32)

        def acc_body(l, accs):
            return tuple(
                accs[j] + vals_t[l, pl.ds(j * _LANES, _LANES)]
                for j in range(groups)
            )

        accs = lax.fori_loop(0, hist, acc_body, (zero,) * groups)
        for j in range(groups):
            y = 1.0 / (1.0 + jnp.exp(-accs[j]))
            y_v[pl.ds(j * _LANES, _LANES)] = _round4(y)
        pltpu.sync_copy(y_v, out_hbm.at[pl.ds(wid * b_per, b_per)])

    return run(x_T, s_flat)


def kernel(x, emb_table, W, b):
    batch, hist = x.shape
    V, _ = emb_table.shape
    scores = _scores(emb_table, W, b, hist)
    x_T = x.T.astype(jnp.int32)
    out = _pool(x_T, scores.reshape(V), batch, hist)
    return out.reshape(batch, 1)


# speedup vs baseline: 18.6348x; 2.5374x over previous
import functools

import jax
import jax.numpy as jnp
from jax import lax
from jax.experimental import pallas as pl
from jax.experimental.pallas import tpu as pltpu
from jax.experimental.pallas import tpu_sc as plsc

_LANES = 16
_VOCAB_BLK = 5120


def _scores_body(w_ref, b_ref, t_ref, o_ref, *, inv_hist):
    t = t_ref[...]
    w = w_ref[...]
    s = jnp.sum(t * w, axis=0)
    o_ref[...] = (s + b_ref[0]) * inv_hist


def _scores(table_t, W_t, b, hist):
    D, V = table_t.shape
    grid = (V + _VOCAB_BLK - 1) // _VOCAB_BLK
    return pl.pallas_call(
        functools.partial(_scores_body, inv_hist=1.0 / float(hist)),
        grid=(grid,),
        in_specs=[
            pl.BlockSpec((D, 1), lambda i: (0, 0)),
            pl.BlockSpec(memory_space=pltpu.SMEM),
            pl.BlockSpec((D, _VOCAB_BLK), lambda i: (0, i)),
        ],
        out_specs=pl.BlockSpec((_VOCAB_BLK,), lambda i: (i,)),
        out_shape=jax.ShapeDtypeStruct((V,), jnp.float32),
    )(W_t, b, table_t)


def _round4(y):
    t = y * 10000.0
    t = (t + 8388608.0) - 8388608.0
    return t / 10000.0


def _pool(x_T, s_flat, batch, hist):
    info = plsc.get_sparse_core_info()
    nc, ns = info.num_cores, info.num_subcores
    nw = nc * ns
    b_per = batch // nw
    groups = b_per // _LANES

    mesh = plsc.VectorSubcoreMesh(core_axis_name="c", subcore_axis_name="s")

    @functools.partial(
        pl.kernel,
        out_type=jax.ShapeDtypeStruct((batch,), jnp.float32),
        mesh=mesh,
        scratch_types=[
            pltpu.VMEM((hist, b_per), jnp.int32),
            pltpu.VMEM((hist, b_per), jnp.float32),
            pltpu.VMEM((b_per,), jnp.float32),
            pltpu.SemaphoreType.DMA,
        ],
    )
    def run(xT_hbm, s_hbm, out_hbm, idx_t, vals_t, y_v, sem):
        wid = lax.axis_index("s") * nc + lax.axis_index("c")
        base = wid * b_per
        pltpu.sync_copy(xT_hbm.at[:, pl.ds(base, b_per)], idx_t)

        n_chunk = 10

        def gather_chunk(k, carry):
            descs = [
                pltpu.async_copy(
                    s_hbm.at[idx_t.at[k * n_chunk + c]],
                    vals_t.at[k * n_chunk + c],
                    sem,
                )
                for c in range(n_chunk)
            ]
            for d in descs:
                d.wait()
            return carry

        lax.fori_loop(0, hist // n_chunk, gather_chunk, 0)

        zero = jnp.zeros((_LANES,), jnp.float32)

        def acc_body(l, accs):
            return tuple(
                accs[j] + vals_t[l, pl.ds(j * _LANES, _LANES)]
                for j in range(groups)
            )

        accs = lax.fori_loop(0, hist, acc_body, (zero,) * groups)
        for j in range(groups):
            y = 1.0 / (1.0 + jnp.exp(-accs[j]))
            y_v[pl.ds(j * _LANES, _LANES)] = _round4(y)
        pltpu.sync_copy(y_v, out_hbm.at[pl.ds(wid * b_per, b_per)])

    return run(x_T, s_flat)


def kernel(x, emb_table, W, b):
    batch, hist = x.shape
    V, _ = emb_table.shape
    scores = _scores(emb_table.T, W.T, b, hist)
    x_T = x.T.astype(jnp.int32)
    out = _pool(x_T, scores.reshape(V), batch, hist)
    return out.reshape(batch, 1)


# speedup vs baseline: 19.2775x; 1.0345x over previous
import functools

import jax
import jax.numpy as jnp
from jax import lax
from jax.experimental import pallas as pl
from jax.experimental.pallas import tpu as pltpu
from jax.experimental.pallas import tpu_sc as plsc

_LANES = 16
_VOCAB_BLK = 5120


def _scores_body(w_ref, b_ref, t_ref, o_ref, *, inv_hist):
    t = t_ref[...]
    w = w_ref[...]
    s = lax.dot_general(w, t, (((0,), (0,)), ((), ())),
                        preferred_element_type=jnp.float32)
    o_ref[...] = (s[0] + b_ref[0]) * inv_hist


def _scores(table_t, W_t, b, hist):
    D, V = table_t.shape
    grid = (V + _VOCAB_BLK - 1) // _VOCAB_BLK
    return pl.pallas_call(
        functools.partial(_scores_body, inv_hist=1.0 / float(hist)),
        grid=(grid,),
        in_specs=[
            pl.BlockSpec((D, 1), lambda i: (0, 0)),
            pl.BlockSpec(memory_space=pltpu.SMEM),
            pl.BlockSpec((D, _VOCAB_BLK), lambda i: (0, i)),
        ],
        out_specs=pl.BlockSpec((_VOCAB_BLK,), lambda i: (i,)),
        out_shape=jax.ShapeDtypeStruct((V,), jnp.float32),
    )(W_t, b, table_t)


def _round4(y):
    t = y * 10000.0
    t = (t + 8388608.0) - 8388608.0
    return t / 10000.0


def _pool(x_T, s_flat, batch, hist):
    info = plsc.get_sparse_core_info()
    nc, ns = info.num_cores, info.num_subcores
    nw = nc * ns
    b_per = batch // nw
    groups = b_per // _LANES

    mesh = plsc.VectorSubcoreMesh(core_axis_name="c", subcore_axis_name="s")

    @functools.partial(
        pl.kernel,
        out_type=jax.ShapeDtypeStruct((batch,), jnp.float32),
        mesh=mesh,
        scratch_types=[
            pltpu.VMEM((hist, b_per), jnp.int32),
            pltpu.VMEM((hist * b_per,), jnp.float32),
            pltpu.VMEM((b_per,), jnp.float32),
            pltpu.SemaphoreType.DMA,
        ],
    )
    def run(xT_hbm, s_hbm, out_hbm, idx_t, vals_v, y_v, sem):
        wid = lax.axis_index("s") * nc + lax.axis_index("c")
        base = wid * b_per
        pltpu.sync_copy(xT_hbm.at[:, pl.ds(base, b_per)], idx_t)

        def fire(l, carry):
            pltpu.async_copy(
                s_hbm.at[idx_t.at[l]],
                vals_v.at[pl.ds(l * b_per, b_per)],
                sem,
            )
            return carry

        lax.fori_loop(0, hist, fire, 0)
        pltpu.make_async_copy(
            s_hbm.at[pl.ds(0, hist * b_per)], vals_v, sem
        ).wait()

        zero = jnp.zeros((_LANES,), jnp.float32)

        def acc_body(l, accs):
            return tuple(
                accs[j] + vals_v[pl.ds(l * b_per + j * _LANES, _LANES)]
                for j in range(groups)
            )

        accs = lax.fori_loop(0, hist, acc_body, (zero,) * groups)
        for j in range(groups):
            y = 1.0 / (1.0 + jnp.exp(-accs[j]))
            y_v[pl.ds(j * _LANES, _LANES)] = _round4(y)
        pltpu.sync_copy(y_v, out_hbm.at[pl.ds(wid * b_per, b_per)])

    return run(x_T, s_flat)


def kernel(x, emb_table, W, b):
    batch, hist = x.shape
    V, _ = emb_table.shape
    scores = _scores(emb_table.T, W.T, b, hist)
    x_T = x.T.astype(jnp.int32)
    out = _pool(x_T, scores.reshape(V), batch, hist)
    return out.reshape(batch, 1)


# speedup vs baseline: 25.2471x; 1.3097x over previous
import functools

import jax
import jax.numpy as jnp
from jax import lax
from jax.experimental import pallas as pl
from jax.experimental.pallas import tpu as pltpu
from jax.experimental.pallas import tpu_sc as plsc

_LANES = 16
_VOCAB_BLK = 25600


def _scores_body(w_ref, b_ref, t_ref, o_ref, *, inv_hist):
    t = t_ref[...]
    w = w_ref[...]
    s = lax.dot_general(w, t, (((1,), (0,)), ((), ())),
                        preferred_element_type=jnp.float32)
    o_ref[...] = (s[0] + b_ref[0]) * inv_hist


def _scores(table_t, W, b, hist):
    D, V = table_t.shape
    grid = (V + _VOCAB_BLK - 1) // _VOCAB_BLK
    return pl.pallas_call(
        functools.partial(_scores_body, inv_hist=1.0 / float(hist)),
        grid=(grid,),
        in_specs=[
            pl.BlockSpec((1, D), lambda i: (0, 0)),
            pl.BlockSpec(memory_space=pltpu.SMEM),
            pl.BlockSpec((D, _VOCAB_BLK), lambda i: (0, i)),
        ],
        out_specs=pl.BlockSpec((_VOCAB_BLK,), lambda i: (i,)),
        out_shape=jax.ShapeDtypeStruct((V,), jnp.float32),
    )(W, b, table_t)


def _round4(y):
    t = y * 10000.0
    t = (t + 8388608.0) - 8388608.0
    return t / 10000.0


def _pool(x_T, s_flat, batch, hist):
    info = plsc.get_sparse_core_info()
    nc, ns = info.num_cores, info.num_subcores
    nw = nc * ns
    b_per = batch // nw
    groups = b_per // _LANES

    mesh = plsc.VectorSubcoreMesh(core_axis_name="c", subcore_axis_name="s")

    @functools.partial(
        pl.kernel,
        out_type=jax.ShapeDtypeStruct((batch,), jnp.float32),
        mesh=mesh,
        scratch_types=[
            pltpu.VMEM((hist, b_per), jnp.int32),
            pltpu.VMEM((hist * b_per,), jnp.float32),
            pltpu.VMEM((b_per,), jnp.float32),
            pltpu.SemaphoreType.DMA,
        ],
    )
    def run(xT_hbm, s_hbm, out_hbm, idx_t, vals_v, y_v, sem):
        wid = lax.axis_index("s") * nc + lax.axis_index("c")
        base = wid * b_per
        pltpu.sync_copy(xT_hbm.at[:, pl.ds(base, b_per)], idx_t)

        def fire(l, carry):
            pltpu.async_copy(
                s_hbm.at[idx_t.at[l]],
                vals_v.at[pl.ds(l * b_per, b_per)],
                sem,
            )
            return carry

        lax.fori_loop(0, hist, fire, 0)
        pltpu.make_async_copy(
            s_hbm.at[pl.ds(0, hist * b_per)], vals_v, sem
        ).wait()

        zero = jnp.zeros((_LANES,), jnp.float32)

        def acc_body(l, accs):
            return tuple(
                accs[j] + vals_v[pl.ds(l * b_per + j * _LANES, _LANES)]
                for j in range(groups)
            )

        accs = lax.fori_loop(0, hist, acc_body, (zero,) * groups)
        for j in range(groups):
            y = 1.0 / (1.0 + jnp.exp(-accs[j]))
            y_v[pl.ds(j * _LANES, _LANES)] = _round4(y)
        pltpu.sync_copy(y_v, out_hbm.at[pl.ds(wid * b_per, b_per)])

    return run(x_T, s_flat)


def kernel(x, emb_table, W, b):
    batch, hist = x.shape
    V, _ = emb_table.shape
    scores = _scores(emb_table.T, W, b, hist)
    x_T = x.T.astype(jnp.int32)
    out = _pool(x_T, scores.reshape(V), batch, hist)
    return out.reshape(batch, 1)


# speedup vs baseline: 25.9286x; 1.0270x over previous
import functools

import jax
import jax.numpy as jnp
from jax import lax
from jax.experimental import pallas as pl
from jax.experimental.pallas import tpu as pltpu
from jax.experimental.pallas import tpu_sc as plsc

_LANES = 16
_VOCAB_BLK = 25600


def _scores_body(w_ref, b_ref, t_ref, o_ref, *, inv_hist):
    t = t_ref[...]
    w = w_ref[...]
    s = lax.dot_general(w, t, (((1,), (0,)), ((), ())),
                        preferred_element_type=jnp.float32)
    o_ref[...] = (s[0] + b_ref[0]) * inv_hist


def _scores(table_t, W, b, hist):
    D, V = table_t.shape
    return pl.pallas_call(
        functools.partial(_scores_body, inv_hist=1.0 / float(hist)),
        in_specs=[
            pl.BlockSpec(memory_space=pltpu.VMEM),
            pl.BlockSpec(memory_space=pltpu.SMEM),
            pl.BlockSpec(memory_space=pltpu.VMEM),
        ],
        out_specs=pl.BlockSpec(memory_space=pltpu.VMEM),
        out_shape=jax.ShapeDtypeStruct((V,), jnp.float32),
    )(W, b, table_t)


def _round4(y):
    t = y * 10000.0
    t = (t + 8388608.0) - 8388608.0
    return t / 10000.0


def _pool(x_T, s_flat, batch, hist):
    info = plsc.get_sparse_core_info()
    nc, ns = info.num_cores, info.num_subcores
    nw = nc * ns
    b_per = batch // nw
    groups = b_per // _LANES

    mesh = plsc.VectorSubcoreMesh(core_axis_name="c", subcore_axis_name="s")

    @functools.partial(
        pl.kernel,
        out_type=jax.ShapeDtypeStruct((batch,), jnp.float32),
        mesh=mesh,
        scratch_types=[
            pltpu.VMEM((hist, b_per), jnp.int32),
            pltpu.VMEM((hist * b_per,), jnp.float32),
            pltpu.VMEM((b_per,), jnp.float32),
            pltpu.SemaphoreType.DMA,
        ],
    )
    def run(xT_hbm, s_hbm, out_hbm, idx_t, vals_v, y_v, sem):
        wid = lax.axis_index("s") * nc + lax.axis_index("c")
        base = wid * b_per
        pltpu.sync_copy(xT_hbm.at[:, pl.ds(base, b_per)], idx_t)

        def fire(l, carry):
            pltpu.async_copy(
                s_hbm.at[idx_t.at[l]],
                vals_v.at[pl.ds(l * b_per, b_per)],
                sem,
            )
            return carry

        lax.fori_loop(0, hist, fire, 0)
        pltpu.make_async_copy(
            s_hbm.at[pl.ds(0, hist * b_per)], vals_v, sem
        ).wait()

        zero = jnp.zeros((_LANES,), jnp.float32)

        def acc_body(l, accs):
            return tuple(
                accs[j] + vals_v[pl.ds(l * b_per + j * _LANES, _LANES)]
                for j in range(groups)
            )

        accs = lax.fori_loop(0, hist, acc_body, (zero,) * groups)
        for j in range(groups):
            y = 1.0 / (1.0 + jnp.exp(-accs[j]))
            y_v[pl.ds(j * _LANES, _LANES)] = _round4(y)
        pltpu.sync_copy(y_v, out_hbm.at[pl.ds(wid * b_per, b_per)])

    return run(x_T, s_flat)


def kernel(x, emb_table, W, b):
    batch, hist = x.shape
    V, _ = emb_table.shape
    scores = _scores(emb_table.T, W, b, hist)
    x_T = x.T.astype(jnp.int32)
    out = _pool(x_T, scores.reshape(V), batch, hist)
    return out.reshape(batch, 1)


# speedup vs baseline: 31.6855x; 1.2220x over previous
import functools

import jax
import jax.numpy as jnp
from jax import lax
from jax.experimental import pallas as pl
from jax.experimental.pallas import tpu as pltpu
from jax.experimental.pallas import tpu_sc as plsc

_LANES = 16
_VOCAB_BLK = 25600


def _scores_body(w_ref, b_ref, t_ref, o_ref, *, inv_hist):
    t = t_ref[...]
    w = w_ref[...]
    s = lax.dot_general(w, t, (((1,), (0,)), ((), ())),
                        preferred_element_type=jnp.float32)
    o_ref[...] = (s[0] + b_ref[0]) * inv_hist


def _scores(table_t, W, b, hist):
    D, V = table_t.shape
    return pl.pallas_call(
        functools.partial(_scores_body, inv_hist=1.0 / float(hist)),
        in_specs=[
            pl.BlockSpec(memory_space=pltpu.VMEM),
            pl.BlockSpec(memory_space=pltpu.SMEM),
            pl.BlockSpec(memory_space=pltpu.VMEM),
        ],
        out_specs=pl.BlockSpec(memory_space=pltpu.VMEM),
        out_shape=jax.ShapeDtypeStruct((V,), jnp.float32),
    )(W, b, table_t)


def _round4(y):
    t = y * 10000.0
    t = (t + 8388608.0) - 8388608.0
    return t / 10000.0


def _pool(x_T, s_flat, batch, hist):
    info = plsc.get_sparse_core_info()
    nc, ns = info.num_cores, info.num_subcores
    nw = nc * ns
    b_per = batch // nw
    groups = b_per // _LANES
    len_s = s_flat.shape[0]

    mesh = plsc.VectorSubcoreMesh(core_axis_name="c", subcore_axis_name="s")

    @functools.partial(
        pl.kernel,
        out_type=jax.ShapeDtypeStruct((batch,), jnp.float32),
        mesh=mesh,
        scratch_types=[
            pltpu.VMEM((hist, b_per), jnp.int32),
            pltpu.VMEM((hist * b_per,), jnp.float32),
            pltpu.VMEM((b_per,), jnp.float32),
            pltpu.VMEM_SHARED((len_s,), jnp.float32),
            pltpu.SemaphoreType.DMA,
        ],
    )
    def run(xT_hbm, s_hbm, out_hbm, idx_t, vals_v, y_v, s_sh, sem):
        sid = lax.axis_index("s")
        wid = sid * nc + lax.axis_index("c")
        base = wid * b_per

        @pl.when(sid == 0)
        def _():
            pltpu.sync_copy(s_hbm, s_sh)

        pltpu.sync_copy(xT_hbm.at[:, pl.ds(base, b_per)], idx_t)
        plsc.subcore_barrier()

        def fire(l, carry):
            pltpu.async_copy(
                s_sh.at[idx_t.at[l]],
                vals_v.at[pl.ds(l * b_per, b_per)],
                sem,
            )
            return carry

        lax.fori_loop(0, hist, fire, 0)
        pltpu.make_async_copy(
            s_hbm.at[pl.ds(0, hist * b_per)], vals_v, sem
        ).wait()

        zero = jnp.zeros((_LANES,), jnp.float32)

        def acc_body(l, accs):
            return tuple(
                accs[j] + vals_v[pl.ds(l * b_per + j * _LANES, _LANES)]
                for j in range(groups)
            )

        accs = lax.fori_loop(0, hist, acc_body, (zero,) * groups)
        for j in range(groups):
            y = 1.0 / (1.0 + jnp.exp(-accs[j]))
            y_v[pl.ds(j * _LANES, _LANES)] = _round4(y)
        pltpu.sync_copy(y_v, out_hbm.at[pl.ds(wid * b_per, b_per)])

    return run(x_T, s_flat)


def kernel(x, emb_table, W, b):
    batch, hist = x.shape
    V, _ = emb_table.shape
    scores = _scores(emb_table.T, W, b, hist)
    x_T = x.T.astype(jnp.int32)
    out = _pool(x_T, scores.reshape(V), batch, hist)
    return out.reshape(batch, 1)
